# baseline (speedup 1.0000x reference)
import functools

import jax
import jax.numpy as jnp
import numpy as np
from jax import lax
from jax.experimental import pallas as pl
from jax.experimental.pallas import tpu as pltpu
from jax.experimental.pallas import tpu_sc as plsc

N_NODES = 10000
N_PAD = 10240
D_IN = 128
D_H = 64
D_OUT = 16

NC = 2
NS = 16
NW = NC * NS
CHUNK = 128
RPT = N_PAD // NS

_f32 = jnp.float32
_mesh = plsc.VectorSubcoreMesh(core_axis_name="c", subcore_axis_name="s")
_sc_params = pltpu.CompilerParams(use_tc_tiling_on_sc=False)


def _worker_split(wid, base, extra):
    start = base * wid + jnp.minimum(wid, extra)
    limit = base + jnp.where(wid < extra, 1, 0)
    return start, limit


def _load_chunks(src, start, extra_pred, base, idx_v):
    pltpu.sync_copy(src.at[pl.ds(start, base)], idx_v.at[pl.ds(0, base)])

    @pl.when(extra_pred)
    def _():
        pltpu.sync_copy(src.at[pl.ds(start + base, 1)],
                        idx_v.at[pl.ds(base, 1)])


def _deg_body(base, extra, dst_hbm, ones_hbm, zero_hbm, out_hbm,
              dst_v, ones_v, acc, sem):
    c = lax.axis_index("c")
    s = lax.axis_index("s")
    wid = c * NS + s
    start, limit = _worker_split(wid, base, extra)
    rows = pl.ds(s * RPT, RPT)
    pltpu.sync_copy(zero_hbm.at[rows], acc.at[rows])
    pltpu.sync_copy(ones_hbm, ones_v)
    _load_chunks(dst_hbm, start, wid < extra, base, dst_v)
    plsc.subcore_barrier()

    def fire(j, carry):
        @pl.when(j < limit)
        def _():
            pltpu.async_copy(ones_v, acc.at[dst_v.at[j]], sem, add=True)
        return carry

    lax.fori_loop(0, base + 1, fire, 0)

    def drain(j, carry):
        @pl.when(j < limit)
        def _():
            pltpu.make_async_copy(ones_v, acc.at[dst_v.at[0]], sem).wait()
        return carry

    lax.fori_loop(0, base + 1, drain, 0)
    plsc.subcore_barrier()
    pltpu.sync_copy(acc.at[rows], out_hbm.at[c].at[rows])


def _sc_degree(dst_rows, base, extra):
    kern = functools.partial(
        pl.kernel,
        out_type=jax.ShapeDtypeStruct((NC, N_PAD, 16), _f32),
        mesh=_mesh,
        scratch_types=[
            pltpu.VMEM((base + 1, CHUNK), jnp.int32),
            pltpu.VMEM((CHUNK, 16), _f32),
            pltpu.VMEM_SHARED((N_PAD, 16), _f32),
            pltpu.SemaphoreType.DMA,
        ],
        compiler_params=_sc_params,
    )(functools.partial(_deg_body, base, extra))
    ones = jnp.ones((CHUNK, 16), _f32)
    zeros = jnp.zeros((N_PAD, 16), _f32)
    return kern(dst_rows, ones, zeros)


def _agg_body(base, extra, d, g_hbm, src_hbm, dst_hbm, zero_hbm, out_hbm,
              src_v, dst_v, b0, b1, b2, b3,
              acc, sg0, sg1, sg2, sg3, ss0, ss1, ss2, ss3):
    bufs = (b0, b1, b2, b3)
    sgs = (sg0, sg1, sg2, sg3)
    sss = (ss0, ss1, ss2, ss3)
    c = lax.axis_index("c")
    s = lax.axis_index("s")
    wid = c * NS + s
    start, limit = _worker_split(wid, base, extra)
    rows = pl.ds(s * RPT, RPT)
    pltpu.sync_copy(zero_hbm.at[rows], acc.at[rows])
    _load_chunks(src_hbm, start, wid < extra, base, src_v)
    _load_chunks(dst_hbm, start, wid < extra, base, dst_v)
    plsc.subcore_barrier()

    mi = max(0, -(-(base - 3) // 4))
    main = 4 * mi

    for b in range(4):
        pltpu.async_copy(g_hbm.at[src_v.at[b]], bufs[b], sgs[b])

    def body(i, carry):
        for b in range(4):
            cb = 4 * i + b
            pltpu.make_async_copy(g_hbm.at[src_v.at[cb]], bufs[b],
                                  sgs[b]).wait()
            pltpu.async_copy(bufs[b], acc.at[dst_v.at[cb]], sss[b], add=True)
        for b in range(4):
            nb = 4 * i + 4 + b
            pltpu.make_async_copy(bufs[b], acc.at[dst_v.at[0]], sss[b]).wait()

            @pl.when(nb < limit)
            def _():
                pltpu.async_copy(g_hbm.at[src_v.at[nb]], bufs[b], sgs[b])
        return carry

    lax.fori_loop(0, mi, body, 0)

    for cb in range(main, base + 1):
        b = cb % 4

        @pl.when(cb < limit)
        def _(cb=cb, b=b):
            pltpu.make_async_copy(g_hbm.at[src_v.at[cb]], bufs[b],
                                  sgs[b]).wait()
            pltpu.sync_copy(bufs[b], acc.at[dst_v.at[cb]], add=True)
    plsc.subcore_barrier()
    pltpu.sync_copy(acc.at[rows], out_hbm.at[c].at[rows])


def _sc_aggregate(g, src_rows, dst_rows, base, extra, d):
    kern = functools.partial(
        pl.kernel,
        out_type=jax.ShapeDtypeStruct((NC, N_PAD, d), _f32),
        mesh=_mesh,
        scratch_types=[
            pltpu.VMEM((base + 1, CHUNK), jnp.int32),
            pltpu.VMEM((base + 1, CHUNK), jnp.int32),
            pltpu.VMEM((CHUNK, d), _f32),
            pltpu.VMEM((CHUNK, d), _f32),
            pltpu.VMEM((CHUNK, d), _f32),
            pltpu.VMEM((CHUNK, d), _f32),
            pltpu.VMEM_SHARED((N_PAD, d), _f32),
            pltpu.SemaphoreType.DMA,
            pltpu.SemaphoreType.DMA,
            pltpu.SemaphoreType.DMA,
            pltpu.SemaphoreType.DMA,
            pltpu.SemaphoreType.DMA,
            pltpu.SemaphoreType.DMA,
            pltpu.SemaphoreType.DMA,
            pltpu.SemaphoreType.DMA,
        ],
        compiler_params=_sc_params,
    )(functools.partial(_agg_body, base, extra, d))
    zeros = jnp.zeros((N_PAD, d), _f32)
    return kern(g, src_rows, dst_rows, zeros)


def _dinv_of(degp_ref):
    p = degp_ref[...]
    deg = p[0, :, 0:1] + p[1, :, 0:1] + 1.0
    return lax.rsqrt(deg)


def _matmul1_body(x_ref, w1_ref, h1_ref):
    h1_ref[...] = jnp.dot(x_ref[...], w1_ref[...],
                          preferred_element_type=_f32)


def _scale1_body(h1_ref, degp_ref, g1_ref):
    g1_ref[...] = h1_ref[...] * _dinv_of(degp_ref)


def _layer2_body(aggp_ref, g1_ref, degp_ref, b1_ref, w2_ref, g2_ref):
    dinv = _dinv_of(degp_ref)
    agg = aggp_ref[0] + aggp_ref[1] + g1_ref[...]
    out1 = agg * dinv + b1_ref[...]
    r = jnp.maximum(out1, 0.0)
    h2 = jnp.dot(r, w2_ref[...], preferred_element_type=_f32)
    g2_ref[...] = h2 * dinv


def _final_body(aggp_ref, g2_ref, degp_ref, b2_ref, out_ref):
    dinv = _dinv_of(degp_ref)
    agg = aggp_ref[0] + aggp_ref[1] + g2_ref[...]
    out_ref[...] = agg * dinv + b2_ref[...]


_TCG = 16
_TCB = N_PAD // _TCG


def _tc_matmul1(x_pad, W1):
    return pl.pallas_call(
        _matmul1_body,
        grid=(_TCG,),
        in_specs=[
            pl.BlockSpec((_TCB, D_IN), lambda i: (i, 0)),
            pl.BlockSpec((D_IN, D_H), lambda i: (0, 0)),
        ],
        out_specs=pl.BlockSpec((_TCB, D_H), lambda i: (i, 0)),
        out_shape=jax.ShapeDtypeStruct((N_PAD, D_H), _f32),
    )(x_pad, W1)


def _tc_scale1(h1, degp):
    return pl.pallas_call(
        _scale1_body,
        grid=(_TCG,),
        in_specs=[
            pl.BlockSpec((_TCB, D_H), lambda i: (i, 0)),
            pl.BlockSpec((NC, _TCB, 16), lambda i: (0, i, 0)),
        ],
        out_specs=pl.BlockSpec((_TCB, D_H), lambda i: (i, 0)),
        out_shape=jax.ShapeDtypeStruct((N_PAD, D_H), _f32),
    )(h1, degp)


def _tc_layer2(aggp1, g1, degp, b1, W2):
    return pl.pallas_call(
        _layer2_body,
        grid=(_TCG,),
        in_specs=[
            pl.BlockSpec((NC, _TCB, D_H), lambda i: (0, i, 0)),
            pl.BlockSpec((_TCB, D_H), lambda i: (i, 0)),
            pl.BlockSpec((NC, _TCB, 16), lambda i: (0, i, 0)),
            pl.BlockSpec((1, D_H), lambda i: (0, 0)),
            pl.BlockSpec((D_H, D_OUT), lambda i: (0, 0)),
        ],
        out_specs=pl.BlockSpec((_TCB, D_OUT), lambda i: (i, 0)),
        out_shape=jax.ShapeDtypeStruct((N_PAD, D_OUT), _f32),
    )(aggp1, g1, degp, b1, W2)


def _tc_final(aggp2, g2, degp, b2, n, blk):
    grid = n // blk
    return pl.pallas_call(
        _final_body,
        grid=(grid,),
        in_specs=[
            pl.BlockSpec((NC, blk, D_OUT), lambda i: (0, i, 0)),
            pl.BlockSpec((blk, D_OUT), lambda i: (i, 0)),
            pl.BlockSpec((NC, blk, 16), lambda i: (0, i, 0)),
            pl.BlockSpec((1, D_OUT), lambda i: (0, 0)),
        ],
        out_specs=pl.BlockSpec((blk, D_OUT), lambda i: (i, 0)),
        out_shape=jax.ShapeDtypeStruct((n, D_OUT), _f32),
    )(aggp2, g2, degp, b2)


def kernel(x, edge_index, W1, b1, W2, b2):
    n, e = x.shape[0], edge_index.shape[1]
    src_1d, dst_1d = edge_index[0], edge_index[1]
    if e % CHUNK or e // CHUNK < 4 * NW:
        e_pad = max(-(-e // CHUNK), 4 * NW) * CHUNK
        pad = jnp.asarray(N_NODES + (np.arange(e_pad - e, dtype=np.int32)
                                     % (N_PAD - N_NODES)))
        src_1d = jnp.concatenate([src_1d, pad])
        dst_1d = jnp.concatenate([dst_1d, pad])
        e = e_pad
    n_chunks = e // CHUNK
    base, extra = n_chunks // NW, n_chunks % NW
    dst_rows = dst_1d.reshape(n_chunks, CHUNK)
    (src_1d,) = lax.optimization_barrier((src_1d,))
    src_rows = src_1d.reshape(n_chunks, CHUNK)

    x_pad = jnp.zeros((N_PAD, D_IN), _f32).at[:n].set(x)

    degp = _sc_degree(dst_rows, base, extra)
    h1 = _tc_matmul1(x_pad, W1)
    g1 = _tc_scale1(h1, degp)
    aggp1 = _sc_aggregate(g1, src_rows, dst_rows, base, extra, D_H)
    g2 = _tc_layer2(aggp1, g1, degp, b1.reshape(1, D_H), W2)
    aggp2 = _sc_aggregate(g2, src_rows, dst_rows, base, extra, D_OUT)
    if n % 1000 == 0:
        return _tc_final(aggp2, g2, degp, b2.reshape(1, D_OUT), n, 1000)
    out = _tc_final(aggp2, g2, degp, b2.reshape(1, D_OUT), N_PAD, _TCB)
    return out[:n]


# speedup vs baseline: 1.0471x; 1.0471x over previous
import functools

import jax
import jax.numpy as jnp
import numpy as np
from jax import lax
from jax.experimental import pallas as pl
from jax.experimental.pallas import tpu as pltpu
from jax.experimental.pallas import tpu_sc as plsc

N_NODES = 10000
N_PAD = 10240
D_IN = 128
D_H = 64
D_OUT = 16

NC = 2
NS = 16
NW = NC * NS
CHUNK = 128
RPT = N_PAD // NS

_f32 = jnp.float32
_mesh = plsc.VectorSubcoreMesh(core_axis_name="c", subcore_axis_name="s")
_sc_params = pltpu.CompilerParams(use_tc_tiling_on_sc=False)


def _worker_split(wid, base, extra):
    start = base * wid + jnp.minimum(wid, extra)
    limit = base + jnp.where(wid < extra, 1, 0)
    return start, limit


def _load_chunks(src, start, extra_pred, base, idx_v):
    pltpu.sync_copy(src.at[pl.ds(start, base)], idx_v.at[pl.ds(0, base)])

    @pl.when(extra_pred)
    def _():
        pltpu.sync_copy(src.at[pl.ds(start + base, 1)],
                        idx_v.at[pl.ds(base, 1)])


def _deg_body(base, extra, dst_hbm, ones_hbm, zero_hbm, out_hbm,
              dst_v, ones_v, acc, sem):
    c = lax.axis_index("c")
    s = lax.axis_index("s")
    wid = c * NS + s
    start, limit = _worker_split(wid, base, extra)
    rows = pl.ds(s * RPT, RPT)
    pltpu.sync_copy(zero_hbm.at[rows], acc.at[rows])
    pltpu.sync_copy(ones_hbm, ones_v)
    _load_chunks(dst_hbm, start, wid < extra, base, dst_v)
    plsc.subcore_barrier()

    def fire(j, carry):
        @pl.when(j < limit)
        def _():
            pltpu.async_copy(ones_v, acc.at[dst_v.at[j]], sem, add=True)
        return carry

    lax.fori_loop(0, base + 1, fire, 0)

    def drain(j, carry):
        @pl.when(j < limit)
        def _():
            pltpu.make_async_copy(ones_v, acc.at[dst_v.at[0]], sem).wait()
        return carry

    lax.fori_loop(0, base + 1, drain, 0)
    plsc.subcore_barrier()
    pltpu.sync_copy(acc.at[rows], out_hbm.at[c].at[rows])


def _sc_degree(dst_rows, base, extra):
    kern = functools.partial(
        pl.kernel,
        out_type=jax.ShapeDtypeStruct((NC, N_PAD, 16), _f32),
        mesh=_mesh,
        scratch_types=[
            pltpu.VMEM((base + 1, CHUNK), jnp.int32),
            pltpu.VMEM((CHUNK, 16), _f32),
            pltpu.VMEM_SHARED((N_PAD, 16), _f32),
            pltpu.SemaphoreType.DMA,
        ],
        compiler_params=_sc_params,
    )(functools.partial(_deg_body, base, extra))
    ones = jnp.ones((CHUNK, 16), _f32)
    zeros = jnp.zeros((N_PAD, 16), _f32)
    return kern(dst_rows, ones, zeros)


def _agg_body(base, extra, d, g_hbm, src_hbm, dst_hbm, zero_hbm, out_hbm,
              src_v, dst_v, b0, b1, b2, b3,
              acc, sg0, sg1, sg2, sg3, ss0, ss1, ss2, ss3):
    bufs = (b0, b1, b2, b3)
    sgs = (sg0, sg1, sg2, sg3)
    sss = (ss0, ss1, ss2, ss3)
    c = lax.axis_index("c")
    s = lax.axis_index("s")
    wid = c * NS + s
    start, limit = _worker_split(wid, base, extra)
    rows = pl.ds(s * RPT, RPT)
    pltpu.sync_copy(zero_hbm.at[rows], acc.at[rows])
    _load_chunks(src_hbm, start, wid < extra, base, src_v)
    _load_chunks(dst_hbm, start, wid < extra, base, dst_v)
    plsc.subcore_barrier()

    mi = max(0, -(-(base - 3) // 4))
    main = 4 * mi

    for b in range(4):
        pltpu.async_copy(g_hbm.at[src_v.at[b]], bufs[b], sgs[b])

    def body(i, carry):
        for b in range(4):
            cb = 4 * i + b
            pltpu.make_async_copy(g_hbm.at[src_v.at[cb]], bufs[b],
                                  sgs[b]).wait()
            pltpu.async_copy(bufs[b], acc.at[dst_v.at[cb]], sss[b], add=True)
        for b in range(4):
            nb = 4 * i + 4 + b
            pltpu.make_async_copy(bufs[b], acc.at[dst_v.at[0]], sss[b]).wait()

            @pl.when(nb < limit)
            def _():
                pltpu.async_copy(g_hbm.at[src_v.at[nb]], bufs[b], sgs[b])
        return carry

    lax.fori_loop(0, mi, body, 0)

    for cb in range(main, base + 1):
        b = cb % 4

        @pl.when(cb < limit)
        def _(cb=cb, b=b):
            pltpu.make_async_copy(g_hbm.at[src_v.at[cb]], bufs[b],
                                  sgs[b]).wait()
            pltpu.sync_copy(bufs[b], acc.at[dst_v.at[cb]], add=True)
    plsc.subcore_barrier()
    pltpu.sync_copy(acc.at[rows], out_hbm.at[c].at[rows])


def _sc_aggregate(g, src_rows, dst_rows, base, extra, d):
    kern = functools.partial(
        pl.kernel,
        out_type=jax.ShapeDtypeStruct((NC, N_PAD, d), _f32),
        mesh=_mesh,
        scratch_types=[
            pltpu.VMEM((base + 1, CHUNK), jnp.int32),
            pltpu.VMEM((base + 1, CHUNK), jnp.int32),
            pltpu.VMEM((CHUNK, d), _f32),
            pltpu.VMEM((CHUNK, d), _f32),
            pltpu.VMEM((CHUNK, d), _f32),
            pltpu.VMEM((CHUNK, d), _f32),
            pltpu.VMEM_SHARED((N_PAD, d), _f32),
            pltpu.SemaphoreType.DMA,
            pltpu.SemaphoreType.DMA,
            pltpu.SemaphoreType.DMA,
            pltpu.SemaphoreType.DMA,
            pltpu.SemaphoreType.DMA,
            pltpu.SemaphoreType.DMA,
            pltpu.SemaphoreType.DMA,
            pltpu.SemaphoreType.DMA,
        ],
        compiler_params=_sc_params,
    )(functools.partial(_agg_body, base, extra, d))
    zeros = jnp.zeros((N_PAD, d), _f32)
    return kern(g, src_rows, dst_rows, zeros)


def _dinv_of(degp_ref):
    p = degp_ref[...]
    deg = p[0, :, 0:1] + p[1, :, 0:1] + 1.0
    return lax.rsqrt(deg)


def _matmul1_body(x_ref, w1_ref, h1_ref):
    h1_ref[...] = jnp.dot(x_ref[...], w1_ref[...],
                          preferred_element_type=_f32)


def _scale1_body(h1_ref, degp_ref, g1_ref):
    g1_ref[...] = h1_ref[...] * _dinv_of(degp_ref)


def _layer2_body(aggp_ref, g1_ref, degp_ref, b1_ref, w2_ref, g2_ref):
    dinv = _dinv_of(degp_ref)
    agg = aggp_ref[0] + aggp_ref[1] + g1_ref[...]
    out1 = agg * dinv + b1_ref[...]
    r = jnp.maximum(out1, 0.0)
    h2 = jnp.dot(r, w2_ref[...], preferred_element_type=_f32)
    g2_ref[...] = h2 * dinv


def _final_body(aggp_ref, g2_ref, degp_ref, b2_ref, out_ref):
    dinv = _dinv_of(degp_ref)
    agg = aggp_ref[0] + aggp_ref[1] + g2_ref[...]
    out_ref[...] = agg * dinv + b2_ref[...]


_TCG = 8
_TCB = N_PAD // _TCG


def _tc_matmul1(x_pad, W1):
    return pl.pallas_call(
        _matmul1_body,
        grid=(_TCG,),
        in_specs=[
            pl.BlockSpec((_TCB, D_IN), lambda i: (i, 0)),
            pl.BlockSpec((D_IN, D_H), lambda i: (0, 0)),
        ],
        out_specs=pl.BlockSpec((_TCB, D_H), lambda i: (i, 0)),
        out_shape=jax.ShapeDtypeStruct((N_PAD, D_H), _f32),
    )(x_pad, W1)


def _tc_scale1(h1, degp):
    return pl.pallas_call(
        _scale1_body,
        grid=(_TCG,),
        in_specs=[
            pl.BlockSpec((_TCB, D_H), lambda i: (i, 0)),
            pl.BlockSpec((NC, _TCB, 16), lambda i: (0, i, 0)),
        ],
        out_specs=pl.BlockSpec((_TCB, D_H), lambda i: (i, 0)),
        out_shape=jax.ShapeDtypeStruct((N_PAD, D_H), _f32),
    )(h1, degp)


def _tc_layer2(aggp1, g1, degp, b1, W2):
    return pl.pallas_call(
        _layer2_body,
        grid=(_TCG,),
        in_specs=[
            pl.BlockSpec((NC, _TCB, D_H), lambda i: (0, i, 0)),
            pl.BlockSpec((_TCB, D_H), lambda i: (i, 0)),
            pl.BlockSpec((NC, _TCB, 16), lambda i: (0, i, 0)),
            pl.BlockSpec((1, D_H), lambda i: (0, 0)),
            pl.BlockSpec((D_H, D_OUT), lambda i: (0, 0)),
        ],
        out_specs=pl.BlockSpec((_TCB, D_OUT), lambda i: (i, 0)),
        out_shape=jax.ShapeDtypeStruct((N_PAD, D_OUT), _f32),
    )(aggp1, g1, degp, b1, W2)


def _tc_final(aggp2, g2, degp, b2, n, blk):
    grid = n // blk
    return pl.pallas_call(
        _final_body,
        grid=(grid,),
        in_specs=[
            pl.BlockSpec((NC, blk, D_OUT), lambda i: (0, i, 0)),
            pl.BlockSpec((blk, D_OUT), lambda i: (i, 0)),
            pl.BlockSpec((NC, blk, 16), lambda i: (0, i, 0)),
            pl.BlockSpec((1, D_OUT), lambda i: (0, 0)),
        ],
        out_specs=pl.BlockSpec((blk, D_OUT), lambda i: (i, 0)),
        out_shape=jax.ShapeDtypeStruct((n, D_OUT), _f32),
    )(aggp2, g2, degp, b2)


def kernel(x, edge_index, W1, b1, W2, b2):
    n, e = x.shape[0], edge_index.shape[1]
    src_1d, dst_1d = edge_index[0], edge_index[1]
    if e % CHUNK or e // CHUNK < 4 * NW:
        e_pad = max(-(-e // CHUNK), 4 * NW) * CHUNK
        pad = jnp.asarray(N_NODES + (np.arange(e_pad - e, dtype=np.int32)
                                     % (N_PAD - N_NODES)))
        src_1d = jnp.concatenate([src_1d, pad])
        dst_1d = jnp.concatenate([dst_1d, pad])
        e = e_pad
    n_chunks = e // CHUNK
    base, extra = n_chunks // NW, n_chunks % NW
    dst_rows = dst_1d.reshape(n_chunks, CHUNK)
    (src_1d,) = lax.optimization_barrier((src_1d,))
    src_rows = src_1d.reshape(n_chunks, CHUNK)

    x_pad = jnp.zeros((N_PAD, D_IN), _f32).at[:n].set(x)

    degp = _sc_degree(dst_rows, base, extra)
    h1 = _tc_matmul1(x_pad, W1)
    g1 = _tc_scale1(h1, degp)
    aggp1 = _sc_aggregate(g1, src_rows, dst_rows, base, extra, D_H)
    g2 = _tc_layer2(aggp1, g1, degp, b1.reshape(1, D_H), W2)
    aggp2 = _sc_aggregate(g2, src_rows, dst_rows, base, extra, D_OUT)
    if n % 1000 == 0:
        return _tc_final(aggp2, g2, degp, b2.reshape(1, D_OUT), n, 1000)
    out = _tc_final(aggp2, g2, degp, b2.reshape(1, D_OUT), N_PAD, _TCB)
    return out[:n]
